# baseline (speedup 1.0000x reference)
import jax
import jax.numpy as jnp
from jax.experimental import pallas as pl
from jax.experimental.pallas import tpu as pltpu

K = 32


def _knn_idx(x, k):
    inner = -2.0 * jnp.einsum('bcn,bcm->bnm', x, x)
    xx = jnp.sum(x * x, axis=1, keepdims=True)
    pair = -jnp.transpose(xx, (0, 2, 1)) - inner - xx
    return jax.lax.top_k(pair, k)[1]


def _get_graph_feature(x, k):
    B, C, N = x.shape
    idx = _knn_idx(x, k)
    xt = jnp.transpose(x, (0, 2, 1))
    nb = jnp.take_along_axis(xt, idx.reshape(B, N * k, 1), axis=1).reshape(B, N, k, C)
    xi = jnp.broadcast_to(xt[:, :, None, :], (B, N, k, C))
    f = jnp.concatenate([nb - xi, xi], axis=-1)
    return jnp.transpose(f, (0, 3, 1, 2))


def _bn2d(x, eps=1e-5):
    m = jnp.mean(x, axis=(0, 2, 3), keepdims=True)
    v = jnp.var(x, axis=(0, 2, 3), keepdims=True)
    return (x - m) / jnp.sqrt(v + eps)


def _bn1d(x, eps=1e-5):
    m = jnp.mean(x, axis=(0, 2), keepdims=True)
    v = jnp.var(x, axis=(0, 2), keepdims=True)
    return (x - m) / jnp.sqrt(v + eps)


def _lrelu(x):
    return jax.nn.leaky_relu(x, 0.2)


def _conv2(x, W):
    return jnp.einsum('bcnk,oc->bonk', x, W)


def _conv1(x, W):
    return jnp.einsum('bcn,oc->bon', x, W)


def _index_select(xyz, feat, n_out, Ws):
    scores = jnp.einsum('bcn,c->bn', feat, Ws)
    values, idx = jax.lax.top_k(scores, n_out)
    gate = jax.nn.sigmoid(values)
    nf = jnp.take_along_axis(feat, idx[:, None, :], axis=2) * gate[:, None, :]
    ns = jnp.take_along_axis(xyz, idx[:, None, :], axis=2)
    return nf, values, idx, scores, ns, ns


def _aggregate(xyz, nodes, feat, k):
    inner = -2.0 * jnp.einsum('bcm,bcn->bmn', nodes, xyz)
    d = -jnp.sum(nodes * nodes, axis=1)[:, :, None] - inner - jnp.sum(xyz * xyz, axis=1)[:, None, :]
    idx = jax.lax.top_k(d, k)[1]
    B, M, _ = idx.shape
    ft = jnp.transpose(feat, (0, 2, 1))
    C = ft.shape[-1]
    nb = jnp.take_along_axis(ft, idx.reshape(B, M * k, 1), axis=1).reshape(B, M, k, C)
    return jnp.transpose(jnp.max(nb, axis=2), (0, 2, 1))


def _head_kernel(g_ref, l1_ref, l2_ref, l2b_ref, l3_ref, l3b_ref, out_ref):
    g = g_ref[...]
    eps = 1e-5
    y = g @ l1_ref[...]
    m = jnp.mean(y, axis=0, keepdims=True)
    v = jnp.mean((y - m) * (y - m), axis=0, keepdims=True)
    y = (y - m) / jnp.sqrt(v + eps)
    y = jnp.where(y > 0, y, 0.2 * y)
    y = y @ l2_ref[...] + l2b_ref[...]
    m = jnp.mean(y, axis=0, keepdims=True)
    v = jnp.mean((y - m) * (y - m), axis=0, keepdims=True)
    y = (y - m) / jnp.sqrt(v + eps)
    y = jnp.where(y > 0, y, 0.2 * y)
    y = y @ l3_ref[...] + l3b_ref[...]
    out_ref[...] = y


def _head(g, L1, L2, L2b, L3, L3b):
    B = g.shape[0]
    return pl.pallas_call(
        _head_kernel,
        out_shape=jax.ShapeDtypeStruct((B, L3.shape[1]), jnp.float32),
    )(g, L1, L2, L2b.reshape(1, -1), L3, L3b.reshape(1, -1))


def kernel(x, W1, W2, W2m, W2p, W3, W4, W4m, W4p, W5, W6, W6m, W6p, W7, W8,
           Ws1, Ws2, Ws3, L1, L2, L2b, L3, L3b):
    k = K
    xyz = x
    h = _lrelu(_bn2d(_conv2(_get_graph_feature(x, k), W1)))
    x1 = jnp.max(h, axis=-1)
    h = _lrelu(_bn2d(_conv2(_get_graph_feature(x1, k), W2)))
    x2 = jnp.max(h, axis=-1)
    x_t1 = _lrelu(_bn1d(_conv1(x2, W2m)))
    nf, v1, idx1, ret1, ns1, n1 = _index_select(xyz, x2, 256, Ws1)
    agg = _aggregate(xyz, ns1, x2, k)
    x_p1 = _lrelu(_bn1d(_conv1(jnp.concatenate([nf, agg], axis=1), W2p)))
    h = _lrelu(_bn2d(_conv2(_get_graph_feature(x_p1, k // 2), W3)))
    x3 = jnp.max(h, axis=-1)
    h = _lrelu(_bn2d(_conv2(_get_graph_feature(x3, k // 2), W4)))
    x4 = _lrelu(jnp.max(h, axis=-1) + x_p1)
    x_t2 = _lrelu(_bn1d(_conv1(x4, W4m)))
    nf, v2, idx2, ret2, ns2, n2 = _index_select(ns1, x4, 64, Ws2)
    agg = _aggregate(ns1, ns2, x4, k // 2)
    x_p2 = _lrelu(_bn1d(_conv1(jnp.concatenate([nf, agg], axis=1), W4p)))
    h = _lrelu(_bn2d(_conv2(_get_graph_feature(x_p2, k // 4), W5)))
    x5 = jnp.max(h, axis=-1)
    h = _lrelu(_bn2d(_conv2(_get_graph_feature(x5, k // 4), W6)))
    x6 = _lrelu(jnp.max(h, axis=-1) + x_p2)
    x_t3 = _lrelu(_bn1d(_conv1(x6, W6m)))
    nf, v3, idx3, ret3, ns3, n3 = _index_select(ns2, x6, 16, Ws3)
    agg = _aggregate(ns2, ns3, x6, k)
    x_p3 = _lrelu(_bn1d(_conv1(jnp.concatenate([nf, agg], axis=1), W6p)))
    h = _lrelu(_bn2d(_conv2(_get_graph_feature(x_p3, k // 8), W7)))
    x7 = jnp.max(h, axis=-1)
    h = _lrelu(_bn2d(_conv2(_get_graph_feature(x7, k // 8), W8)))
    x_t4 = _lrelu(jnp.max(h, axis=-1) + x_p3)
    g = jnp.concatenate([jnp.max(x_t1, -1), jnp.max(x_t2, -1),
                         jnp.max(x_t3, -1), jnp.max(x_t4, -1)], axis=1)
    y = _head(g, L1, L2, L2b, L3, L3b)
    return (y, ret1, ret2, ret3, n1, n2, n3, ns1, ns2, ns3)


# speedup vs baseline: 1.0004x; 1.0004x over previous
import functools

import jax
import jax.numpy as jnp
from jax import lax
from jax.experimental import pallas as pl

_EPS = 1e-5
_K = 32


def _lrelu(x):
    return jnp.maximum(x, 0.2 * x)


def _knn_idx(x, k):
    inner = -2.0 * jnp.einsum('bcn,bcm->bnm', x, x)
    xx = jnp.sum(x * x, axis=1, keepdims=True)
    pair = -jnp.transpose(xx, (0, 2, 1)) - inner - xx
    return jax.lax.top_k(pair, k)[1]


def _graph_feature(x, k):
    b, c, n = x.shape
    idx = _knn_idx(x, k)
    xt = jnp.transpose(x, (0, 2, 1))
    nb = jnp.take_along_axis(xt, idx.reshape(b, n * k, 1), axis=1).reshape(b, n, k, c)
    xi = jnp.broadcast_to(xt[:, :, None, :], (b, n, k, c))
    f = jnp.concatenate([nb - xi, xi], axis=-1)
    return jnp.transpose(f, (0, 3, 1, 2))


def _bn2d(x):
    m = jnp.mean(x, axis=(0, 2, 3), keepdims=True)
    v = jnp.var(x, axis=(0, 2, 3), keepdims=True)
    return (x - m) / jnp.sqrt(v + _EPS)


def _bn1d(x):
    m = jnp.mean(x, axis=(0, 2), keepdims=True)
    v = jnp.var(x, axis=(0, 2), keepdims=True)
    return (x - m) / jnp.sqrt(v + _EPS)


def _conv2(x, w):
    return jnp.einsum('bcnk,oc->bonk', x, w)


def _conv1(x, w):
    return jnp.einsum('bcn,oc->bon', x, w)


def _sel_pool(xyz, feat, n_out, ws):
    scores = jnp.einsum('bcn,c->bn', feat, ws)
    values, idx = jax.lax.top_k(scores, n_out)
    gate = jax.nn.sigmoid(values)
    nf = jnp.take_along_axis(feat, idx[:, None, :], axis=2) * gate[:, None, :]
    ns = jnp.take_along_axis(xyz, idx[:, None, :], axis=2)
    return nf, scores, ns


def _aggregate(xyz, nodes, feat, k):
    inner = -2.0 * jnp.einsum('bcm,bcn->bmn', nodes, xyz)
    d = (-jnp.sum(nodes * nodes, axis=1)[:, :, None] - inner
         - jnp.sum(xyz * xyz, axis=1)[:, None, :])
    idx = jax.lax.top_k(d, k)[1]
    b, m, _ = idx.shape
    ft = jnp.transpose(feat, (0, 2, 1))
    c = ft.shape[-1]
    nb = jnp.take_along_axis(ft, idx.reshape(b, m * k, 1), axis=1).reshape(b, m, k, c)
    return jnp.transpose(jnp.max(nb, axis=2), (0, 2, 1))


def _ct_kernel(x_ref, wm_ref, ctm_ref, cts_ref, ctss_ref):
    xf = x_ref[0]
    c = lax.dot_general(xf, wm_ref[...], (((1,), (1,)), ((), ())),
                        preferred_element_type=jnp.float32)
    ctm_ref[...] = jnp.max(c, axis=0).reshape(1, 1, -1)
    cts_ref[...] = jnp.sum(c, axis=0).reshape(1, 1, -1)
    ctss_ref[...] = jnp.sum(c * c, axis=0).reshape(1, 1, -1)


def _ct_branch(x_bcn, wm):
    x_nc = jnp.transpose(x_bcn, (0, 2, 1))
    b, n, c = x_nc.shape
    om = wm.shape[0]
    ctm, cts, ctss = pl.pallas_call(
        _ct_kernel,
        grid=(b,),
        in_specs=[
            pl.BlockSpec((1, n, c), lambda i: (i, 0, 0)),
            pl.BlockSpec((om, c), lambda i: (0, 0)),
        ],
        out_specs=[
            pl.BlockSpec((1, 1, om), lambda i: (i, 0, 0)),
            pl.BlockSpec((1, 1, om), lambda i: (i, 0, 0)),
            pl.BlockSpec((1, 1, om), lambda i: (i, 0, 0)),
        ],
        out_shape=[
            jax.ShapeDtypeStruct((b, 1, om), jnp.float32),
            jax.ShapeDtypeStruct((b, 1, om), jnp.float32),
            jax.ShapeDtypeStruct((b, 1, om), jnp.float32),
        ],
    )(x_nc, wm)
    return ctm[:, 0, :], cts[:, 0, :], ctss[:, 0, :]


def _head_kernel(c1m_ref, c1s_ref, c1ss_ref,
                 c2m_ref, c2s_ref, c2ss_ref,
                 c3m_ref, c3s_ref, c3ss_ref, g4_ref,
                 l1_ref, l2_ref, l2b_ref, l3_ref, l3b_ref, out_ref,
                 *, cnt1, cnt2, cnt3):

    def gpart(cm, cs, css, cnt):
        m = jnp.sum(cs, axis=0, keepdims=True) / cnt
        v = jnp.sum(css, axis=0, keepdims=True) / cnt - m * m
        return _lrelu((cm - m) / jnp.sqrt(v + _EPS))

    g1 = gpart(c1m_ref[...], c1s_ref[...], c1ss_ref[...], cnt1)
    g2 = gpart(c2m_ref[...], c2s_ref[...], c2ss_ref[...], cnt2)
    g3 = gpart(c3m_ref[...], c3s_ref[...], c3ss_ref[...], cnt3)
    g = jnp.concatenate([g1, g2, g3, g4_ref[...]], axis=1)

    def bn0d_lrelu(y):
        m = jnp.mean(y, axis=0, keepdims=True)
        v = jnp.mean((y - m) * (y - m), axis=0, keepdims=True)
        return _lrelu((y - m) / jnp.sqrt(v + _EPS))

    y = bn0d_lrelu(g @ l1_ref[...])
    y = bn0d_lrelu(y @ l2_ref[...] + l2b_ref[...])
    out_ref[...] = y @ l3_ref[...] + l3b_ref[...]


def _head(ct1, ct2, ct3, g4, l1, l2, l2b, l3, l3b, *, cnt1, cnt2, cnt3):
    b = g4.shape[0]
    kern = functools.partial(_head_kernel, cnt1=cnt1, cnt2=cnt2, cnt3=cnt3)
    return pl.pallas_call(
        kern,
        out_shape=jax.ShapeDtypeStruct((b, l3.shape[1]), jnp.float32),
    )(*ct1, *ct2, *ct3, g4, l1, l2, l2b.reshape(1, -1), l3, l3b.reshape(1, -1))


def kernel(x, W1, W2, W2m, W2p, W3, W4, W4m, W4p, W5, W6, W6m, W6p, W7, W8,
           Ws1, Ws2, Ws3, L1, L2, L2b, L3, L3b):
    k = _K
    b = x.shape[0]
    xyz = x
    h = _lrelu(_bn2d(_conv2(_graph_feature(x, k), W1)))
    x1 = jnp.max(h, axis=-1)
    h = _lrelu(_bn2d(_conv2(_graph_feature(x1, k), W2)))
    x2 = jnp.max(h, axis=-1)
    ct1 = _ct_branch(x2, W2m)
    nf, ret1, ns1 = _sel_pool(xyz, x2, 256, Ws1)
    agg = _aggregate(xyz, ns1, x2, k)
    x_p1 = _lrelu(_bn1d(_conv1(jnp.concatenate([nf, agg], axis=1), W2p)))
    h = _lrelu(_bn2d(_conv2(_graph_feature(x_p1, k // 2), W3)))
    x3 = jnp.max(h, axis=-1)
    h = _lrelu(_bn2d(_conv2(_graph_feature(x3, k // 2), W4)))
    x4 = _lrelu(jnp.max(h, axis=-1) + x_p1)
    ct2 = _ct_branch(x4, W4m)
    nf, ret2, ns2 = _sel_pool(ns1, x4, 64, Ws2)
    agg = _aggregate(ns1, ns2, x4, k // 2)
    x_p2 = _lrelu(_bn1d(_conv1(jnp.concatenate([nf, agg], axis=1), W4p)))
    h = _lrelu(_bn2d(_conv2(_graph_feature(x_p2, k // 4), W5)))
    x5 = jnp.max(h, axis=-1)
    h = _lrelu(_bn2d(_conv2(_graph_feature(x5, k // 4), W6)))
    x6 = _lrelu(jnp.max(h, axis=-1) + x_p2)
    ct3 = _ct_branch(x6, W6m)
    nf, ret3, ns3 = _sel_pool(ns2, x6, 16, Ws3)
    agg = _aggregate(ns2, ns3, x6, k)
    x_p3 = _lrelu(_bn1d(_conv1(jnp.concatenate([nf, agg], axis=1), W6p)))
    h = _lrelu(_bn2d(_conv2(_graph_feature(x_p3, k // 8), W7)))
    x7 = jnp.max(h, axis=-1)
    h = _lrelu(_bn2d(_conv2(_graph_feature(x7, k // 8), W8)))
    x_t4 = _lrelu(jnp.max(h, axis=-1) + x_p3)
    g4 = jnp.max(x_t4, axis=-1)
    y = _head(ct1, ct2, ct3, g4, L1, L2, L2b, L3, L3b,
              cnt1=b * 1024, cnt2=b * 256, cnt3=b * 64)
    return (y, ret1, ret2, ret3, ns1, ns2, ns3, ns1, ns2, ns3)
